# baseline (speedup 1.0000x reference)
import functools

import jax
import jax.numpy as jnp
from jax.experimental import pallas as pl
from jax.experimental.pallas import tpu as pltpu

_SMOOTHING = 0.1
_IGNORE_INDEX = 0


def _loss_kernel(pred_ref, tgt_ref, out_ref, m_sc, s_sc, sx_sc, pt_sc,
                 *, vocab, chunk, nj):
    j = pl.program_id(1)
    x = pred_ref[...]
    t = tgt_ref[...]
    col = j * chunk + jax.lax.broadcasted_iota(jnp.int32, x.shape, 1)
    valid = col < vocab
    xm = jnp.where(valid, x, -jnp.inf)
    cmax = jnp.max(xm, axis=-1, keepdims=True)
    csx = jnp.sum(jnp.where(valid, x, 0.0), axis=-1, keepdims=True)
    cpt = jnp.sum(jnp.where(col == t, x, 0.0), axis=-1, keepdims=True)

    first = j == 0
    m_old = jnp.where(first, cmax, m_sc[...])
    m_new = jnp.maximum(m_old, cmax)
    e = jnp.sum(jnp.exp(xm - m_new), axis=-1, keepdims=True)
    s_old = jnp.where(first, 0.0, s_sc[...])
    s_sc[...] = s_old * jnp.exp(m_old - m_new) + e
    m_sc[...] = m_new
    sx_sc[...] = jnp.where(first, 0.0, sx_sc[...]) + csx
    pt_sc[...] = jnp.where(first, 0.0, pt_sc[...]) + cpt

    @pl.when(j == nj - 1)
    def _():
        lse = m_sc[...] + jnp.log(s_sc[...])
        eps = _SMOOTHING / (vocab - 2)
        conf = 1.0 - _SMOOTHING
        loss = -(eps * (sx_sc[...] - vocab * lse)
                 + (conf - eps) * (pt_sc[...] - lse))
        out_ref[...] = jnp.where(t == _IGNORE_INDEX, 0.0, loss)


def kernel(pred, target):
    n, vocab = pred.shape
    r = 32
    if vocab >= 12800:
        chunk = 12800
    else:
        chunk = 128 * ((vocab + 511) // 512)
    nj = (vocab + chunk - 1) // chunk
    tgt = target.astype(jnp.int32).reshape(n, 1)
    acc = pltpu.VMEM((r, 1), jnp.float32)
    row_losses = pl.pallas_call(
        functools.partial(_loss_kernel, vocab=vocab, chunk=chunk, nj=nj),
        grid=(n // r, nj),
        in_specs=[
            pl.BlockSpec((r, chunk), lambda i, j: (i, j)),
            pl.BlockSpec((r, 1), lambda i, j: (i, 0)),
        ],
        out_specs=pl.BlockSpec((r, 1), lambda i, j: (i, 0)),
        out_shape=jax.ShapeDtypeStruct((n, 1), jnp.float32),
        scratch_shapes=[acc, acc, acc, acc],
        compiler_params=pltpu.CompilerParams(
            dimension_semantics=("arbitrary", "arbitrary"),
        ),
    )(pred, tgt)
    return jnp.sum(row_losses) / n
